# Initial kernel scaffold; baseline (speedup 1.0000x reference)
import jax
import jax.numpy as jnp
from jax.experimental import pallas as pl


def kernel(x_path, x_omic1, x_omic2, x_omic3, x_omic4, x_omic5, x_omic6, params):
    raise NotImplementedError("write your pallas kernel here")


# speedup vs baseline: 1.6588x; 1.6588x over previous
import functools

import jax
import jax.numpy as jnp
import numpy as np
from jax.experimental import pallas as pl
from jax.experimental.pallas import tpu as pltpu

D = 256
HEADS = 4
DH = 64
BUCKET = 128
N_HASHES = 2
NEG = -1e9


def _lin(x, p):
    return x @ p['W'] + p['b']


def _layernorm(x, g, b):
    mu = x.mean(-1, keepdims=True)
    var = ((x - mu) ** 2).mean(-1, keepdims=True)
    return (x - mu) / jnp.sqrt(var + 1e-5) * g + b


def _k1_body(nbuck, bn, x_ref, ww_ref, bw_ref, wqk_ref, wv_ref, rm_ref,
             hpb_ref, qkh_ref, vh_ref, bkt_ref):
    x = x_ref[...]
    h = jnp.dot(x, ww_ref[...], preferred_element_type=jnp.float32)
    h = jnp.maximum(h + bw_ref[...], 0.0)
    hpb_ref[...] = h
    qk = jnp.dot(h, wqk_ref[...], preferred_element_type=jnp.float32)
    v = jnp.dot(h, wv_ref[...], preferred_element_type=jnp.float32)
    for hh in range(HEADS):
        sl = slice(hh * DH, (hh + 1) * DH)
        qkh_ref[hh] = qk[:, sl]
        vh_ref[hh] = v[:, sl]
    rotated = jnp.dot(qk, rm_ref[...], preferred_element_type=jnp.float32)
    iota = jax.lax.broadcasted_iota(jnp.int32, (bn, nbuck), 1)
    cols = []
    for s in range(HEADS * N_HASHES):
        seg = rotated[:, s * nbuck:(s + 1) * nbuck]
        mx = jnp.max(seg, axis=-1, keepdims=True)
        idx = jnp.min(jnp.where(seg >= mx, iota, nbuck), axis=-1,
                      keepdims=True)
        cols.append(idx)
    bkt_ref[...] = jnp.concatenate(cols, axis=-1)


def _run_k1(x_path, p, rmat, bn):
    n = x_path.shape[0]
    nbuck = rmat.shape[1] // (HEADS * N_HASHES)
    grid = (n // bn,)
    kfn = functools.partial(_k1_body, nbuck, bn)
    return pl.pallas_call(
        kfn,
        grid=grid,
        in_specs=[
            pl.BlockSpec((bn, x_path.shape[1]), lambda i: (i, 0)),
            pl.BlockSpec((x_path.shape[1], D), lambda i: (0, 0)),
            pl.BlockSpec((1, D), lambda i: (0, 0)),
            pl.BlockSpec((D, D), lambda i: (0, 0)),
            pl.BlockSpec((D, D), lambda i: (0, 0)),
            pl.BlockSpec((D, rmat.shape[1]), lambda i: (0, 0)),
        ],
        out_specs=[
            pl.BlockSpec((bn, D), lambda i: (i, 0)),
            pl.BlockSpec((HEADS, bn, DH), lambda i: (0, i, 0)),
            pl.BlockSpec((HEADS, bn, DH), lambda i: (0, i, 0)),
            pl.BlockSpec((bn, HEADS * N_HASHES), lambda i: (i, 0)),
        ],
        out_shape=[
            jax.ShapeDtypeStruct((n, D), jnp.float32),
            jax.ShapeDtypeStruct((HEADS, n, DH), jnp.float32),
            jax.ShapeDtypeStruct((HEADS, n, DH), jnp.float32),
            jax.ShapeDtypeStruct((n, HEADS * N_HASHES), jnp.int32),
        ],
    )(x_path, p['wsi']['W'], p['wsi']['b'].reshape(1, D),
      p['lsh']['qk'], p['lsh']['v'], rmat)


def _k3_body(sq_ref, sqp_ref, sv_ref, svp_ref, pq_ref, pk_ref, pkp_ref,
             mk_ref, mkp_ref, o_ref, lse_ref):
    cq = sq_ref[0, 0]
    cqp = sqp_ref[0, 0]

    def nrm(t):
        return t / (jnp.sqrt(jnp.sum(t * t, -1, keepdims=True)) + 1e-6)

    ck2 = jnp.concatenate([nrm(cq), nrm(cqp)], axis=0)
    cv2 = jnp.concatenate([sv_ref[0, 0], svp_ref[0, 0]], axis=0)
    dots = jax.lax.dot_general(cq, ck2, (((1,), (1,)), ((), ())),
                               preferred_element_type=jnp.float32)
    dots = dots * (1.0 / np.sqrt(DH))
    pq = pq_ref[0, 0]
    pk = jnp.concatenate([pk_ref[0, 0], pkp_ref[0, 0]], axis=1)
    dots = dots - 1e5 * (pq == pk).astype(jnp.float32)
    mk = jnp.concatenate([mk_ref[0, 0], mkp_ref[0, 0]], axis=1)
    dots = jnp.where(mk != 0, dots, NEG)
    m = jnp.max(dots, -1, keepdims=True)
    ex = jnp.exp(dots - m)
    s = jnp.sum(ex, -1, keepdims=True)
    lse_ref[0, 0] = m + jnp.log(s)
    o_ref[0, 0] = jax.lax.dot_general(ex / s, cv2, (((1,), (0,)), ((), ())),
                                      preferred_element_type=jnp.float32)


def _run_k3(sq, sv, pq, pk, mk, nch):
    rows = sq.shape[0]
    grid = (rows, nch)
    prev = lambda r, i: (r, (i + nch - 1) % nch, 0, 0)
    cur = lambda r, i: (r, i, 0, 0)
    return pl.pallas_call(
        _k3_body,
        grid=grid,
        in_specs=[
            pl.BlockSpec((1, 1, BUCKET, DH), cur),
            pl.BlockSpec((1, 1, BUCKET, DH), prev),
            pl.BlockSpec((1, 1, BUCKET, DH), cur),
            pl.BlockSpec((1, 1, BUCKET, DH), prev),
            pl.BlockSpec((1, 1, BUCKET, 1), cur),
            pl.BlockSpec((1, 1, 1, BUCKET), cur),
            pl.BlockSpec((1, 1, 1, BUCKET), prev),
            pl.BlockSpec((1, 1, 1, BUCKET), cur),
            pl.BlockSpec((1, 1, 1, BUCKET), prev),
        ],
        out_specs=[
            pl.BlockSpec((1, 1, BUCKET, DH), cur),
            pl.BlockSpec((1, 1, BUCKET, 1), cur),
        ],
        out_shape=[
            jax.ShapeDtypeStruct((rows, nch, BUCKET, DH), jnp.float32),
            jax.ShapeDtypeStruct((rows, nch, BUCKET, 1), jnp.float32),
        ],
    )(sq, sq, sv, sv, pq, pk, pk, mk, mk)


def _k4_body(bn,
             o0_ref, o1_ref, l0_ref, l1_ref, hpb_ref, e4_ref,
             wo_ref, bo_ref, gps_ref, bps_ref,
             wa_ref, ba_ref, wb_ref, bb_ref, wc_ref, bc_ref,
             wqp_ref, bqp_ref, wkp_ref, bkp_ref, wvp_ref, bvp_ref,
             wop_ref, bop_ref, gpc_ref, bpc_ref,
             wa2_ref, ba2_ref, wb2_ref, bb2_ref, wc2_ref, bc2_ref,
             hob_ref, qoc_ref, wko_ref, bko_ref, wvo_ref, bvo_ref,
             out_ps_ref, out_pc_ref, out_oc_ref,
             m_ps, l_ps, a_ps, m_pc, l_pc, a_pc, m_oc, l_oc, a_oc):
    i = pl.program_id(0)
    nb = pl.num_programs(0)

    @pl.when(i == 0)
    def _init():
        m_ps[...] = jnp.full_like(m_ps[...], -1e30)
        m_pc[...] = jnp.full_like(m_pc[...], -1e30)
        m_oc[...] = jnp.full_like(m_oc[...], -1e30)
        l_ps[...] = jnp.zeros_like(l_ps[...])
        l_pc[...] = jnp.zeros_like(l_pc[...])
        l_oc[...] = jnp.zeros_like(l_oc[...])
        a_ps[...] = jnp.zeros_like(a_ps[...])
        a_pc[...] = jnp.zeros_like(a_pc[...])
        a_oc[...] = jnp.zeros_like(a_oc[...])

    hpb = hpb_ref[...]

    def mm(a, b):
        return jnp.dot(a, b, preferred_element_type=jnp.float32)

    def pool_update(m_r, l_r, a_r, avec, y):
        mb = jnp.max(avec, axis=0, keepdims=True)
        mn = jnp.maximum(m_r[...], mb)
        alpha = jnp.exp(m_r[...] - mn)
        pv = jnp.exp(avec - mn)
        l_r[...] = alpha * l_r[...] + jnp.sum(pv, axis=0, keepdims=True)
        contrib = jax.lax.dot_general(pv, y, (((0,), (0,)), ((), ())),
                                      preferred_element_type=jnp.float32)
        a_r[...] = alpha * a_r[...] + contrib
        m_r[...] = mn

    def gate_branch(x_res, g_r, b_r, wa, ba, wb, bb, wc, bc):
        y = _layernorm(x_res, g_r[...], b_r[...])
        a = jnp.tanh(mm(y, wa[...]) + ba[...])
        s = jax.nn.sigmoid(mm(y, wb[...]) + bb[...])
        avec = mm(a * s, wc[...]) + bc[...]
        return y, avec

    la = l0_ref[...]
    lb = l1_ref[...]
    mml = jnp.maximum(la, lb)
    z = mml + jnp.log(jnp.exp(la - mml) + jnp.exp(lb - mml))
    w0 = mm(jnp.exp(la - z), e4_ref[...])
    w1 = mm(jnp.exp(lb - z), e4_ref[...])
    merged = w0 * o0_ref[...] + w1 * o1_ref[...]
    att = mm(merged, wo_ref[...]) + bo_ref[...]
    y_ps, a_vec = gate_branch(att + hpb, gps_ref, bps_ref,
                              wa_ref, ba_ref, wb_ref, bb_ref, wc_ref, bc_ref)
    pool_update(m_ps, l_ps, a_ps, a_vec, y_ps)

    kp = mm(hob_ref[...], wkp_ref[...]) + bkp_ref[...]
    vp = mm(hob_ref[...], wvp_ref[...]) + bvp_ref[...]
    q = mm(hpb, wqp_ref[...]) + bqp_ref[...]
    colmask = jax.lax.broadcasted_iota(jnp.int32, (1, 8), 1) < 6
    ohs = []
    for hh in range(HEADS):
        sl = slice(hh * DH, (hh + 1) * DH)
        dots = jax.lax.dot_general(q[:, sl], kp[:, sl],
                                   (((1,), (1,)), ((), ())),
                                   preferred_element_type=jnp.float32)
        dots = dots * (1.0 / np.sqrt(DH))
        dots = jnp.where(colmask, dots, NEG)
        mx = jnp.max(dots, -1, keepdims=True)
        ex = jnp.exp(dots - mx)
        attn = ex / jnp.sum(ex, -1, keepdims=True)
        ohs.append(mm(attn, vp[:, sl]))
    o_pc = mm(jnp.concatenate(ohs, axis=-1), wop_ref[...]) + bop_ref[...]
    y_pc, a_vec2 = gate_branch(o_pc + hpb, gpc_ref, bpc_ref, wa2_ref,
                               ba2_ref, wb2_ref, bb2_ref, wc2_ref, bc2_ref)
    pool_update(m_pc, l_pc, a_pc, a_vec2, y_pc)

    ko = mm(hpb, wko_ref[...]) + bko_ref[...]
    vo = mm(hpb, wvo_ref[...]) + bvo_ref[...]
    qoc = qoc_ref[...]
    for hh in range(HEADS):
        sl = slice(hh * DH, (hh + 1) * DH)
        rs = slice(hh * 8, (hh + 1) * 8)
        st = jax.lax.dot_general(qoc[:, sl], ko[:, sl],
                                 (((1,), (1,)), ((), ())),
                                 preferred_element_type=jnp.float32)
        st = st * (1.0 / np.sqrt(DH))
        mb = jnp.max(st, axis=1, keepdims=True)
        mo = m_oc[rs, :]
        mn = jnp.maximum(mo, mb)
        alpha = jnp.exp(mo - mn)
        pmat = jnp.exp(st - mn)
        l_oc[rs, :] = alpha * l_oc[rs, :] + jnp.sum(pmat, axis=1,
                                                    keepdims=True)
        a_oc[rs, :] = alpha * a_oc[rs, :] + mm(pmat, vo[:, sl])
        m_oc[rs, :] = mn

    @pl.when(i == nb - 1)
    def _fin():
        out_ps_ref[...] = a_ps[...] / l_ps[...]
        out_pc_ref[...] = a_pc[...] / l_pc[...]
        out_oc_ref[...] = a_oc[...] / l_oc[...]


def _run_k4(o0, o1, l0, l1, hpb, e4, p, hob8, qoc8, bn):
    n = hpb.shape[0]
    grid = (n // bn,)
    row = lambda i: (i, 0)
    whole = lambda i: (0, 0)

    def wspec(arr):
        return pl.BlockSpec(arr.shape, whole)

    g = p['ps_gate']
    g2 = p['pc_gate']
    ops = [
        o0, o1, l0, l1, hpb, e4,
        p['lsh']['o']['W'], p['lsh']['o']['b'].reshape(1, D),
        p['ln_ps']['g'].reshape(1, D), p['ln_ps']['b'].reshape(1, D),
        g['a']['W'], g['a']['b'].reshape(1, D),
        g['bgate']['W'], g['bgate']['b'].reshape(1, D),
        g['c']['W'], g['c']['b'].reshape(1, 1),
        p['pc_mha']['q']['W'], p['pc_mha']['q']['b'].reshape(1, D),
        p['pc_mha']['k']['W'], p['pc_mha']['k']['b'].reshape(1, D),
        p['pc_mha']['v']['W'], p['pc_mha']['v']['b'].reshape(1, D),
        p['pc_mha']['o']['W'], p['pc_mha']['o']['b'].reshape(1, D),
        p['ln_pc']['g'].reshape(1, D), p['ln_pc']['b'].reshape(1, D),
        g2['a']['W'], g2['a']['b'].reshape(1, D),
        g2['bgate']['W'], g2['bgate']['b'].reshape(1, D),
        g2['c']['W'], g2['c']['b'].reshape(1, 1),
        hob8, qoc8,
        p['oc_mha']['k']['W'], p['oc_mha']['k']['b'].reshape(1, D),
        p['oc_mha']['v']['W'], p['oc_mha']['v']['b'].reshape(1, D),
    ]
    in_specs = [
        pl.BlockSpec((bn, D), row), pl.BlockSpec((bn, D), row),
        pl.BlockSpec((bn, HEADS), row), pl.BlockSpec((bn, HEADS), row),
        pl.BlockSpec((bn, D), row),
    ] + [wspec(a) for a in ops[5:]]
    kfn = functools.partial(_k4_body, bn)
    return pl.pallas_call(
        kfn,
        grid=grid,
        in_specs=in_specs,
        out_specs=[
            pl.BlockSpec((1, D), whole),
            pl.BlockSpec((1, D), whole),
            pl.BlockSpec((HEADS * 8, DH), whole),
        ],
        out_shape=[
            jax.ShapeDtypeStruct((1, D), jnp.float32),
            jax.ShapeDtypeStruct((1, D), jnp.float32),
            jax.ShapeDtypeStruct((HEADS * 8, DH), jnp.float32),
        ],
        scratch_shapes=[
            pltpu.VMEM((1, 1), jnp.float32), pltpu.VMEM((1, 1), jnp.float32),
            pltpu.VMEM((1, D), jnp.float32),
            pltpu.VMEM((1, 1), jnp.float32), pltpu.VMEM((1, 1), jnp.float32),
            pltpu.VMEM((1, D), jnp.float32),
            pltpu.VMEM((HEADS * 8, 1), jnp.float32),
            pltpu.VMEM((HEADS * 8, 1), jnp.float32),
            pltpu.VMEM((HEADS * 8, DH), jnp.float32),
        ],
    )(*ops)


def kernel(x_path, x_omic1, x_omic2, x_omic3, x_omic4, x_omic5, x_omic6,
           params):
    p = params
    n = x_path.shape[0]
    padlen = 2 * BUCKET - n % (2 * BUCKET)
    t = n + padlen
    nch = t // BUCKET
    nbuck = t // BUCKET

    rot = jax.random.normal(jax.random.key(42), (DH, N_HASHES, nbuck // 2))
    rhr = jnp.concatenate([rot, -rot], axis=-1).reshape(DH, N_HASHES * nbuck)
    rmat = jnp.kron(jnp.eye(HEADS, dtype=jnp.float32), rhr)

    bn = 512 if n % 512 == 0 else BUCKET
    hpb, qkh, vh, bkt = _run_k1(x_path, p, rmat, bn)

    omics = [x_omic1, x_omic2, x_omic3, x_omic4, x_omic5, x_omic6]
    h_omic = [jax.nn.elu(_lin(jax.nn.elu(_lin(o, s['l0'])), s['l1']))
              for o, s in zip(omics, p['sig'])]
    hob = jnp.stack(h_omic)
    hob8 = jnp.concatenate([hob, jnp.zeros((2, D))], axis=0)
    qoc8 = jnp.concatenate(
        [_lin(hob, p['oc_mha']['q']), jnp.zeros((2, D))], axis=0)

    zpad = jnp.zeros((HEADS, padlen, DH), jnp.float32)
    qk_t = jnp.concatenate([qkh, zpad], axis=1)
    v_t = jnp.concatenate([vh, zpad], axis=1)
    bk = bkt.reshape(n, HEADS, N_HASHES).transpose(2, 1, 0)
    bk = jnp.concatenate(
        [bk, jnp.zeros((N_HASHES, HEADS, padlen), jnp.int32)], axis=-1)
    pos = jnp.arange(t, dtype=jnp.int32)
    keys = (bk * t + pos[None, None, :]).reshape(N_HASHES * HEADS, t)
    sidx = jnp.argsort(keys, axis=-1).astype(jnp.int32)
    qk2 = jnp.concatenate([qk_t, qk_t], axis=0)
    v2 = jnp.concatenate([v_t, v_t], axis=0)
    sq = jnp.take_along_axis(qk2, sidx[..., None], axis=1)
    sv = jnp.take_along_axis(v2, sidx[..., None], axis=1)
    sq = sq.reshape(N_HASHES * HEADS, nch, BUCKET, DH)
    sv = sv.reshape(N_HASHES * HEADS, nch, BUCKET, DH)
    pq = sidx.reshape(N_HASHES * HEADS, nch, BUCKET, 1)
    pk = sidx.reshape(N_HASHES * HEADS, nch, 1, BUCKET)
    mk = (sidx < n).astype(jnp.int32).reshape(N_HASHES * HEADS, nch, 1,
                                              BUCKET)

    o_s, lse_s = _run_k3(sq, sv, pq, pk, mk, nch)

    uidx = jnp.zeros((N_HASHES * HEADS, t), jnp.int32)
    uidx = jnp.put_along_axis(
        uidx, sidx, jnp.broadcast_to(pos[None, :], sidx.shape), axis=1,
        inplace=False)
    o_us = jnp.take_along_axis(o_s.reshape(N_HASHES * HEADS, t, DH),
                               uidx[..., None], axis=1)
    l_us = jnp.take_along_axis(lse_s.reshape(N_HASHES * HEADS, t), uidx,
                               axis=1)
    o_us = o_us.reshape(N_HASHES, HEADS, t, DH)[:, :, :n, :]
    l_us = l_us.reshape(N_HASHES, HEADS, t)[:, :, :n]
    o0 = o_us[0].transpose(1, 0, 2).reshape(n, D)
    o1 = o_us[1].transpose(1, 0, 2).reshape(n, D)
    l0 = l_us[0].transpose(1, 0)
    l1 = l_us[1].transpose(1, 0)

    e4 = jnp.kron(jnp.eye(HEADS, dtype=jnp.float32),
                  jnp.ones((1, DH), jnp.float32))

    pooled_ps, pooled_pc, oc_acc = _run_k4(o0, o1, l0, l1, hpb, e4, p,
                                           hob8, qoc8, bn)

    hps = jax.nn.relu(_lin(pooled_ps, p['ps_rho']))
    hpc = jax.nn.relu(_lin(pooled_pc, p['pc_rho']))

    oc_h = oc_acc.reshape(HEADS, 8, DH)[:, :6, :]
    oc_m = oc_h.transpose(1, 0, 2).reshape(6, D)
    hoc = _lin(oc_m, p['oc_mha']['o'])
    hoc = _layernorm(hoc + hob, p['ln_oc']['g'], p['ln_oc']['b'])
    a = jnp.tanh(_lin(hoc, p['oc_gate']['a']))
    b = jax.nn.sigmoid(_lin(hoc, p['oc_gate']['bgate']))
    av = _lin(a * b, p['oc_gate']['c'])
    hoc = jax.nn.softmax(av.T, 1) @ hoc
    hoc = jax.nn.relu(_lin(hoc, p['oc_rho']))

    def mha6(q, k, v, mp):
        qp = _lin(q, mp['q']); kp = _lin(k, mp['k']); vp = _lin(v, mp['v'])
        qh = qp.reshape(6, HEADS, DH).transpose(1, 0, 2)
        kh = kp.reshape(6, HEADS, DH).transpose(1, 0, 2)
        vh_ = vp.reshape(6, HEADS, DH).transpose(1, 0, 2)
        att = jax.nn.softmax(
            jnp.einsum('hid,hjd->hij', qh, kh) / np.sqrt(DH), -1)
        o = jnp.einsum('hij,hjd->hid', att, vh_).transpose(1, 0, 2)
        return _lin(o.reshape(6, D), mp['o'])

    hos = mha6(hob, hob, hob, p['os_mha'])
    hos = _layernorm(hos + hob, p['ln_os']['g'], p['ln_os']['b'])
    a = jnp.tanh(_lin(hos, p['os_gate']['a']))
    b = jax.nn.sigmoid(_lin(hos, p['os_gate']['bgate']))
    av = _lin(a * b, p['os_gate']['c'])
    hos = jax.nn.softmax(av.T, 1) @ hos
    hos = jax.nn.relu(_lin(hos, p['os_rho']))

    h = jnp.concatenate([hps, hpc, hos, hoc], 0)
    qs = _lin(h, p['query'])
    ks = _lin(h, p['key'])
    aw = jax.nn.softmax(qs @ ks.T, 1)
    hf = (aw @ h).sum(0)
    logits = _lin(hf, p['cls'])[None]
    return (logits, _lin(hps, p['cls_ps']), _lin(hpc, p['cls_pc']),
            _lin(hos, p['cls_os']), _lin(hoc, p['cls_oc']))


# speedup vs baseline: 2.9474x; 1.7768x over previous
import functools

import jax
import jax.numpy as jnp
import numpy as np
from jax.experimental import pallas as pl
from jax.experimental.pallas import tpu as pltpu

D = 256
HEADS = 4
DH = 64
BUCKET = 128
N_HASHES = 2
NEG = -1e9


def _lin(x, p):
    return x @ p['W'] + p['b']


def _layernorm(x, g, b):
    mu = x.mean(-1, keepdims=True)
    var = ((x - mu) ** 2).mean(-1, keepdims=True)
    return (x - mu) / jnp.sqrt(var + 1e-5) * g + b


def _k1_body(nbuck, bn, x_ref, ww_ref, bw_ref, wqk_ref, wv_ref, rm_ref,
             hpb_ref, qkvh_ref, bkt_ref):
    x = x_ref[...]
    h = jnp.dot(x, ww_ref[...], preferred_element_type=jnp.float32)
    h = jnp.maximum(h + bw_ref[...], 0.0)
    hpb_ref[...] = h
    qk = jnp.dot(h, wqk_ref[...], preferred_element_type=jnp.float32)
    v = jnp.dot(h, wv_ref[...], preferred_element_type=jnp.float32)
    for hh in range(HEADS):
        sl = slice(hh * DH, (hh + 1) * DH)
        qkvh_ref[hh] = jnp.concatenate([qk[:, sl], v[:, sl]], axis=-1)
    rotated = jnp.dot(qk, rm_ref[...], preferred_element_type=jnp.float32)
    iota = jax.lax.broadcasted_iota(jnp.int32, (bn, nbuck), 1)
    cols = []
    for s in range(HEADS * N_HASHES):
        seg = rotated[:, s * nbuck:(s + 1) * nbuck]
        mx = jnp.max(seg, axis=-1, keepdims=True)
        idx = jnp.min(jnp.where(seg >= mx, iota, nbuck), axis=-1,
                      keepdims=True)
        cols.append(idx)
    bkt_ref[...] = jnp.concatenate(cols, axis=-1)


def _run_k1(x_path, p, rmat, bn):
    n = x_path.shape[0]
    nbuck = rmat.shape[1] // (HEADS * N_HASHES)
    grid = (n // bn,)
    kfn = functools.partial(_k1_body, nbuck, bn)
    return pl.pallas_call(
        kfn,
        grid=grid,
        in_specs=[
            pl.BlockSpec((bn, x_path.shape[1]), lambda i: (i, 0)),
            pl.BlockSpec((x_path.shape[1], D), lambda i: (0, 0)),
            pl.BlockSpec((1, D), lambda i: (0, 0)),
            pl.BlockSpec((D, D), lambda i: (0, 0)),
            pl.BlockSpec((D, D), lambda i: (0, 0)),
            pl.BlockSpec((D, rmat.shape[1]), lambda i: (0, 0)),
        ],
        out_specs=[
            pl.BlockSpec((bn, D), lambda i: (i, 0)),
            pl.BlockSpec((HEADS, bn, 2 * DH), lambda i: (0, i, 0)),
            pl.BlockSpec((bn, HEADS * N_HASHES), lambda i: (i, 0)),
        ],
        out_shape=[
            jax.ShapeDtypeStruct((n, D), jnp.float32),
            jax.ShapeDtypeStruct((HEADS, n, 2 * DH), jnp.float32),
            jax.ShapeDtypeStruct((n, HEADS * N_HASHES), jnp.int32),
        ],
        compiler_params=pltpu.CompilerParams(
            dimension_semantics=("parallel",)),
    )(x_path, p['wsi']['W'], p['wsi']['b'].reshape(1, D),
      p['lsh']['qk'], p['lsh']['v'], rmat)


def _k3_body(rows, sqv_ref, sqvp_ref, pq_ref, pk_ref, pkp_ref,
             mk_ref, mkp_ref, o_ref, lse_ref):
    def nrm(t):
        return t / (jnp.sqrt(jnp.sum(t * t, -1, keepdims=True)) + 1e-6)

    for hh in range(rows):
        cur = sqv_ref[hh, 0]
        prv = sqvp_ref[hh, 0]
        cq = cur[:, :DH]
        ck2 = jnp.concatenate([nrm(cq), nrm(prv[:, :DH])], axis=0)
        cv2 = jnp.concatenate([cur[:, DH:], prv[:, DH:]], axis=0)
        dots = jax.lax.dot_general(cq, ck2, (((1,), (1,)), ((), ())),
                                   preferred_element_type=jnp.float32)
        dots = dots * (1.0 / np.sqrt(DH))
        pq = pq_ref[hh, 0]
        pk = jnp.concatenate([pk_ref[hh, 0], pkp_ref[hh, 0]], axis=1)
        dots = dots - 1e5 * (pq == pk).astype(jnp.float32)
        mk = jnp.concatenate([mk_ref[hh, 0], mkp_ref[hh, 0]], axis=1)
        dots = jnp.where(mk != 0, dots, NEG)
        m = jnp.max(dots, -1, keepdims=True)
        ex = jnp.exp(dots - m)
        s = jnp.sum(ex, -1, keepdims=True)
        lse_ref[hh, 0] = m + jnp.log(s)
        o_ref[hh, 0] = jax.lax.dot_general(
            ex / s, cv2, (((1,), (0,)), ((), ())),
            preferred_element_type=jnp.float32)


def _run_k3(sqv, pq, pk, mk, nch):
    rows = sqv.shape[0]
    grid = (nch,)
    prev = lambda i: (0, (i + nch - 1) % nch, 0, 0)
    cur = lambda i: (0, i, 0, 0)
    return pl.pallas_call(
        functools.partial(_k3_body, rows),
        grid=grid,
        in_specs=[
            pl.BlockSpec((rows, 1, BUCKET, 2 * DH), cur),
            pl.BlockSpec((rows, 1, BUCKET, 2 * DH), prev),
            pl.BlockSpec((rows, 1, BUCKET, 1), cur),
            pl.BlockSpec((rows, 1, 1, BUCKET), cur),
            pl.BlockSpec((rows, 1, 1, BUCKET), prev),
            pl.BlockSpec((rows, 1, 1, BUCKET), cur),
            pl.BlockSpec((rows, 1, 1, BUCKET), prev),
        ],
        out_specs=[
            pl.BlockSpec((rows, 1, BUCKET, DH), cur),
            pl.BlockSpec((rows, 1, BUCKET, 1), cur),
        ],
        out_shape=[
            jax.ShapeDtypeStruct((rows, nch, BUCKET, DH), jnp.float32),
            jax.ShapeDtypeStruct((rows, nch, BUCKET, 1), jnp.float32),
        ],
        compiler_params=pltpu.CompilerParams(
            dimension_semantics=("parallel",)),
    )(sqv, sqv, pq, pk, pk, mk, mk)


def _k4_body(bn,
             o0_ref, o1_ref, l0_ref, l1_ref, hpb_ref, e4_ref,
             wo_ref, bo_ref, gps_ref, bps_ref,
             wa_ref, ba_ref, wb_ref, bb_ref, wc_ref, bc_ref,
             wqp_ref, bqp_ref, wkp_ref, bkp_ref, wvp_ref, bvp_ref,
             wop_ref, bop_ref, gpc_ref, bpc_ref,
             wa2_ref, ba2_ref, wb2_ref, bb2_ref, wc2_ref, bc2_ref,
             hob_ref, qoc_ref, wko_ref, bko_ref, wvo_ref, bvo_ref,
             out_ps_ref, out_pc_ref, out_oc_ref,
             m_ps, l_ps, a_ps, m_pc, l_pc, a_pc, m_oc, l_oc, a_oc):
    i = pl.program_id(0)
    nb = pl.num_programs(0)

    @pl.when(i == 0)
    def _init():
        m_ps[...] = jnp.full_like(m_ps[...], -1e30)
        m_pc[...] = jnp.full_like(m_pc[...], -1e30)
        m_oc[...] = jnp.full_like(m_oc[...], -1e30)
        l_ps[...] = jnp.zeros_like(l_ps[...])
        l_pc[...] = jnp.zeros_like(l_pc[...])
        l_oc[...] = jnp.zeros_like(l_oc[...])
        a_ps[...] = jnp.zeros_like(a_ps[...])
        a_pc[...] = jnp.zeros_like(a_pc[...])
        a_oc[...] = jnp.zeros_like(a_oc[...])

    hpb = hpb_ref[...]

    def mm(a, b):
        return jnp.dot(a, b, preferred_element_type=jnp.float32)

    def pool_update(m_r, l_r, a_r, avec, y):
        mb = jnp.max(avec, axis=0, keepdims=True)
        mn = jnp.maximum(m_r[...], mb)
        alpha = jnp.exp(m_r[...] - mn)
        pv = jnp.exp(avec - mn)
        l_r[...] = alpha * l_r[...] + jnp.sum(pv, axis=0, keepdims=True)
        contrib = jax.lax.dot_general(pv, y, (((0,), (0,)), ((), ())),
                                      preferred_element_type=jnp.float32)
        a_r[...] = alpha * a_r[...] + contrib
        m_r[...] = mn

    def gate_branch(x_res, g_r, b_r, wa, ba, wb, bb, wc, bc):
        y = _layernorm(x_res, g_r[...], b_r[...])
        a = jnp.tanh(mm(y, wa[...]) + ba[...])
        s = jax.nn.sigmoid(mm(y, wb[...]) + bb[...])
        avec = mm(a * s, wc[...]) + bc[...]
        return y, avec

    la = l0_ref[...]
    lb = l1_ref[...]
    mml = jnp.maximum(la, lb)
    z = mml + jnp.log(jnp.exp(la - mml) + jnp.exp(lb - mml))
    w0 = mm(jnp.exp(la - z), e4_ref[...])
    w1 = mm(jnp.exp(lb - z), e4_ref[...])
    merged = w0 * o0_ref[...] + w1 * o1_ref[...]
    att = mm(merged, wo_ref[...]) + bo_ref[...]
    y_ps, a_vec = gate_branch(att + hpb, gps_ref, bps_ref,
                              wa_ref, ba_ref, wb_ref, bb_ref, wc_ref, bc_ref)
    pool_update(m_ps, l_ps, a_ps, a_vec, y_ps)

    kp = mm(hob_ref[...], wkp_ref[...]) + bkp_ref[...]
    vp = mm(hob_ref[...], wvp_ref[...]) + bvp_ref[...]
    q = mm(hpb, wqp_ref[...]) + bqp_ref[...]
    colmask = jax.lax.broadcasted_iota(jnp.int32, (1, 8), 1) < 6
    ohs = []
    for hh in range(HEADS):
        sl = slice(hh * DH, (hh + 1) * DH)
        dots = jax.lax.dot_general(q[:, sl], kp[:, sl],
                                   (((1,), (1,)), ((), ())),
                                   preferred_element_type=jnp.float32)
        dots = dots * (1.0 / np.sqrt(DH))
        dots = jnp.where(colmask, dots, NEG)
        mx = jnp.max(dots, -1, keepdims=True)
        ex = jnp.exp(dots - mx)
        attn = ex / jnp.sum(ex, -1, keepdims=True)
        ohs.append(mm(attn, vp[:, sl]))
    o_pc = mm(jnp.concatenate(ohs, axis=-1), wop_ref[...]) + bop_ref[...]
    y_pc, a_vec2 = gate_branch(o_pc + hpb, gpc_ref, bpc_ref, wa2_ref,
                               ba2_ref, wb2_ref, bb2_ref, wc2_ref, bc2_ref)
    pool_update(m_pc, l_pc, a_pc, a_vec2, y_pc)

    ko = mm(hpb, wko_ref[...]) + bko_ref[...]
    vo = mm(hpb, wvo_ref[...]) + bvo_ref[...]
    qoc = qoc_ref[...]
    for hh in range(HEADS):
        sl = slice(hh * DH, (hh + 1) * DH)
        rs = slice(hh * 8, (hh + 1) * 8)
        st = jax.lax.dot_general(qoc[:, sl], ko[:, sl],
                                 (((1,), (1,)), ((), ())),
                                 preferred_element_type=jnp.float32)
        st = st * (1.0 / np.sqrt(DH))
        mb = jnp.max(st, axis=1, keepdims=True)
        mo = m_oc[rs, :]
        mn = jnp.maximum(mo, mb)
        alpha = jnp.exp(mo - mn)
        pmat = jnp.exp(st - mn)
        l_oc[rs, :] = alpha * l_oc[rs, :] + jnp.sum(pmat, axis=1,
                                                    keepdims=True)
        a_oc[rs, :] = alpha * a_oc[rs, :] + mm(pmat, vo[:, sl])
        m_oc[rs, :] = mn

    @pl.when(i == nb - 1)
    def _fin():
        out_ps_ref[...] = a_ps[...] / l_ps[...]
        out_pc_ref[...] = a_pc[...] / l_pc[...]
        out_oc_ref[...] = a_oc[...] / l_oc[...]


def _run_k4(o0, o1, l0, l1, hpb, e4, p, hob8, qoc8, bn):
    n = hpb.shape[0]
    grid = (n // bn,)
    row = lambda i: (i, 0)
    whole = lambda i: (0, 0)

    def wspec(arr):
        return pl.BlockSpec(arr.shape, whole)

    g = p['ps_gate']
    g2 = p['pc_gate']
    ops = [
        o0, o1, l0, l1, hpb, e4,
        p['lsh']['o']['W'], p['lsh']['o']['b'].reshape(1, D),
        p['ln_ps']['g'].reshape(1, D), p['ln_ps']['b'].reshape(1, D),
        g['a']['W'], g['a']['b'].reshape(1, D),
        g['bgate']['W'], g['bgate']['b'].reshape(1, D),
        g['c']['W'], g['c']['b'].reshape(1, 1),
        p['pc_mha']['q']['W'], p['pc_mha']['q']['b'].reshape(1, D),
        p['pc_mha']['k']['W'], p['pc_mha']['k']['b'].reshape(1, D),
        p['pc_mha']['v']['W'], p['pc_mha']['v']['b'].reshape(1, D),
        p['pc_mha']['o']['W'], p['pc_mha']['o']['b'].reshape(1, D),
        p['ln_pc']['g'].reshape(1, D), p['ln_pc']['b'].reshape(1, D),
        g2['a']['W'], g2['a']['b'].reshape(1, D),
        g2['bgate']['W'], g2['bgate']['b'].reshape(1, D),
        g2['c']['W'], g2['c']['b'].reshape(1, 1),
        hob8, qoc8,
        p['oc_mha']['k']['W'], p['oc_mha']['k']['b'].reshape(1, D),
        p['oc_mha']['v']['W'], p['oc_mha']['v']['b'].reshape(1, D),
    ]
    in_specs = [
        pl.BlockSpec((bn, D), row), pl.BlockSpec((bn, D), row),
        pl.BlockSpec((bn, HEADS), row), pl.BlockSpec((bn, HEADS), row),
        pl.BlockSpec((bn, D), row),
    ] + [wspec(a) for a in ops[5:]]
    kfn = functools.partial(_k4_body, bn)
    return pl.pallas_call(
        kfn,
        grid=grid,
        in_specs=in_specs,
        out_specs=[
            pl.BlockSpec((1, D), whole),
            pl.BlockSpec((1, D), whole),
            pl.BlockSpec((HEADS * 8, DH), whole),
        ],
        out_shape=[
            jax.ShapeDtypeStruct((1, D), jnp.float32),
            jax.ShapeDtypeStruct((1, D), jnp.float32),
            jax.ShapeDtypeStruct((HEADS * 8, DH), jnp.float32),
        ],
        scratch_shapes=[
            pltpu.VMEM((1, 1), jnp.float32), pltpu.VMEM((1, 1), jnp.float32),
            pltpu.VMEM((1, D), jnp.float32),
            pltpu.VMEM((1, 1), jnp.float32), pltpu.VMEM((1, 1), jnp.float32),
            pltpu.VMEM((1, D), jnp.float32),
            pltpu.VMEM((HEADS * 8, 1), jnp.float32),
            pltpu.VMEM((HEADS * 8, 1), jnp.float32),
            pltpu.VMEM((HEADS * 8, DH), jnp.float32),
        ],
    )(*ops)


def kernel(x_path, x_omic1, x_omic2, x_omic3, x_omic4, x_omic5, x_omic6,
           params):
    p = params
    n = x_path.shape[0]
    padlen = 2 * BUCKET - n % (2 * BUCKET)
    t = n + padlen
    nch = t // BUCKET
    nbuck = t // BUCKET

    rot = jax.random.normal(jax.random.key(42), (DH, N_HASHES, nbuck // 2))
    rhr = jnp.concatenate([rot, -rot], axis=-1).reshape(DH, N_HASHES * nbuck)
    rmat = jnp.kron(jnp.eye(HEADS, dtype=jnp.float32), rhr)

    bn = 512 if n % 512 == 0 else BUCKET
    hpb, qkvh, bkt = _run_k1(x_path, p, rmat, bn)

    omics = [x_omic1, x_omic2, x_omic3, x_omic4, x_omic5, x_omic6]
    h_omic = [jax.nn.elu(_lin(jax.nn.elu(_lin(o, s['l0'])), s['l1']))
              for o, s in zip(omics, p['sig'])]
    hob = jnp.stack(h_omic)
    hob8 = jnp.concatenate([hob, jnp.zeros((2, D))], axis=0)
    qoc8 = jnp.concatenate(
        [_lin(hob, p['oc_mha']['q']), jnp.zeros((2, D))], axis=0)

    zpad = jnp.zeros((HEADS, padlen, 2 * DH), jnp.float32)
    qkv_t = jnp.concatenate([qkvh, zpad], axis=1)
    bk = bkt.reshape(n, HEADS, N_HASHES).transpose(2, 1, 0)
    bk = jnp.concatenate(
        [bk, jnp.zeros((N_HASHES, HEADS, padlen), jnp.int32)], axis=-1)
    pos = jnp.arange(t, dtype=jnp.int32)
    keys = (bk * t + pos[None, None, :]).reshape(N_HASHES * HEADS, t)
    sidx = jnp.argsort(keys, axis=-1).astype(jnp.int32)
    sidx2 = sidx.reshape(N_HASHES, HEADS, t)
    sqv = jnp.stack(
        [jnp.take_along_axis(qkv_t, sidx2[r][..., None], axis=1)
         for r in range(N_HASHES)], axis=0)
    sqv = sqv.reshape(N_HASHES * HEADS, nch, BUCKET, 2 * DH)
    pq = sidx.reshape(N_HASHES * HEADS, nch, BUCKET, 1)
    pk = sidx.reshape(N_HASHES * HEADS, nch, 1, BUCKET)
    mk = (sidx < n).astype(jnp.int32).reshape(N_HASHES * HEADS, nch, 1,
                                              BUCKET)

    o_s, lse_s = _run_k3(sqv, pq, pk, mk, nch)

    uidx = jnp.argsort(sidx, axis=-1).astype(jnp.int32)
    o_us = jnp.take_along_axis(o_s.reshape(N_HASHES * HEADS, t, DH),
                               uidx[..., None], axis=1)
    l_us = jnp.take_along_axis(lse_s.reshape(N_HASHES * HEADS, t), uidx,
                               axis=1)
    o_us = o_us.reshape(N_HASHES, HEADS, t, DH)[:, :, :n, :]
    l_us = l_us.reshape(N_HASHES, HEADS, t)[:, :, :n]
    o0 = o_us[0].transpose(1, 0, 2).reshape(n, D)
    o1 = o_us[1].transpose(1, 0, 2).reshape(n, D)
    l0 = l_us[0].transpose(1, 0)
    l1 = l_us[1].transpose(1, 0)

    e4 = jnp.kron(jnp.eye(HEADS, dtype=jnp.float32),
                  jnp.ones((1, DH), jnp.float32))

    pooled_ps, pooled_pc, oc_acc = _run_k4(o0, o1, l0, l1, hpb, e4, p,
                                           hob8, qoc8, bn)

    hps = jax.nn.relu(_lin(pooled_ps, p['ps_rho']))
    hpc = jax.nn.relu(_lin(pooled_pc, p['pc_rho']))

    oc_h = oc_acc.reshape(HEADS, 8, DH)[:, :6, :]
    oc_m = oc_h.transpose(1, 0, 2).reshape(6, D)
    hoc = _lin(oc_m, p['oc_mha']['o'])
    hoc = _layernorm(hoc + hob, p['ln_oc']['g'], p['ln_oc']['b'])
    a = jnp.tanh(_lin(hoc, p['oc_gate']['a']))
    b = jax.nn.sigmoid(_lin(hoc, p['oc_gate']['bgate']))
    av = _lin(a * b, p['oc_gate']['c'])
    hoc = jax.nn.softmax(av.T, 1) @ hoc
    hoc = jax.nn.relu(_lin(hoc, p['oc_rho']))

    def mha6(q, k, v, mp):
        qp = _lin(q, mp['q']); kp = _lin(k, mp['k']); vp = _lin(v, mp['v'])
        qh = qp.reshape(6, HEADS, DH).transpose(1, 0, 2)
        kh = kp.reshape(6, HEADS, DH).transpose(1, 0, 2)
        vh_ = vp.reshape(6, HEADS, DH).transpose(1, 0, 2)
        att = jax.nn.softmax(
            jnp.einsum('hid,hjd->hij', qh, kh) / np.sqrt(DH), -1)
        o = jnp.einsum('hij,hjd->hid', att, vh_).transpose(1, 0, 2)
        return _lin(o.reshape(6, D), mp['o'])

    hos = mha6(hob, hob, hob, p['os_mha'])
    hos = _layernorm(hos + hob, p['ln_os']['g'], p['ln_os']['b'])
    a = jnp.tanh(_lin(hos, p['os_gate']['a']))
    b = jax.nn.sigmoid(_lin(hos, p['os_gate']['bgate']))
    av = _lin(a * b, p['os_gate']['c'])
    hos = jax.nn.softmax(av.T, 1) @ hos
    hos = jax.nn.relu(_lin(hos, p['os_rho']))

    h = jnp.concatenate([hps, hpc, hos, hoc], 0)
    qs = _lin(h, p['query'])
    ks = _lin(h, p['key'])
    aw = jax.nn.softmax(qs @ ks.T, 1)
    hf = (aw @ h).sum(0)
    logits = _lin(hf, p['cls'])[None]
    return (logits, _lin(hps, p['cls_ps']), _lin(hpc, p['cls_pc']),
            _lin(hos, p['cls_os']), _lin(hoc, p['cls_oc']))


# speedup vs baseline: 3.2661x; 1.1081x over previous
import functools

import jax
import jax.numpy as jnp
import numpy as np
from jax import lax
from jax.experimental import pallas as pl
from jax.experimental.pallas import tpu as pltpu
from jax.experimental.pallas import tpu_sc as plsc

D = 256
HEADS = 4
DH = 64
BUCKET = 128
N_HASHES = 2
NEG = -1e9


def _lin(x, p):
    return x @ p['W'] + p['b']


def _layernorm(x, g, b):
    mu = x.mean(-1, keepdims=True)
    var = ((x - mu) ** 2).mean(-1, keepdims=True)
    return (x - mu) / jnp.sqrt(var + 1e-5) * g + b


def _k1_body(nbuck, bn, x_ref, ww_ref, bw_ref, wqk_ref, wv_ref, rm_ref,
             hpb_ref, qkvh_ref, bkt_ref):
    x = x_ref[...]
    h = jnp.dot(x, ww_ref[...], preferred_element_type=jnp.float32)
    h = jnp.maximum(h + bw_ref[...], 0.0)
    hpb_ref[...] = h
    qk = jnp.dot(h, wqk_ref[...], preferred_element_type=jnp.float32)
    v = jnp.dot(h, wv_ref[...], preferred_element_type=jnp.float32)
    for hh in range(HEADS):
        sl = slice(hh * DH, (hh + 1) * DH)
        qkvh_ref[hh] = jnp.concatenate([qk[:, sl], v[:, sl]], axis=-1)
    rotated = jnp.dot(qk, rm_ref[...], preferred_element_type=jnp.float32)
    iota = jax.lax.broadcasted_iota(jnp.int32, (bn, nbuck), 1)
    cols = []
    for s in range(HEADS * N_HASHES):
        seg = rotated[:, s * nbuck:(s + 1) * nbuck]
        mx = jnp.max(seg, axis=-1, keepdims=True)
        idx = jnp.min(jnp.where(seg >= mx, iota, nbuck), axis=-1,
                      keepdims=True)
        cols.append(idx)
    bkt_ref[...] = jnp.concatenate(cols, axis=-1)


def _run_k1(x_path, p, rmat, bn):
    n = x_path.shape[0]
    nbuck = rmat.shape[1] // (HEADS * N_HASHES)
    grid = (n // bn,)
    kfn = functools.partial(_k1_body, nbuck, bn)
    return pl.pallas_call(
        kfn,
        grid=grid,
        in_specs=[
            pl.BlockSpec((bn, x_path.shape[1]), lambda i: (i, 0)),
            pl.BlockSpec((x_path.shape[1], D), lambda i: (0, 0)),
            pl.BlockSpec((1, D), lambda i: (0, 0)),
            pl.BlockSpec((D, D), lambda i: (0, 0)),
            pl.BlockSpec((D, D), lambda i: (0, 0)),
            pl.BlockSpec((D, rmat.shape[1]), lambda i: (0, 0)),
        ],
        out_specs=[
            pl.BlockSpec((bn, D), lambda i: (i, 0)),
            pl.BlockSpec((HEADS, bn, 2 * DH), lambda i: (0, i, 0)),
            pl.BlockSpec((bn, HEADS * N_HASHES), lambda i: (i, 0)),
        ],
        out_shape=[
            jax.ShapeDtypeStruct((n, D), jnp.float32),
            jax.ShapeDtypeStruct((HEADS, n, 2 * DH), jnp.float32),
            jax.ShapeDtypeStruct((n, HEADS * N_HASHES), jnp.int32),
        ],
        compiler_params=pltpu.CompilerParams(
            dimension_semantics=("parallel",)),
    )(x_path, p['wsi']['W'], p['wsi']['b'].reshape(1, D),
      p['lsh']['qk'], p['lsh']['v'], rmat)


def _sc_gather_rows(table, idx, chunk):
    total = idx.shape[0]
    width = table.shape[1]
    info = plsc.get_sparse_core_info()
    nw = info.num_cores * info.num_subcores
    per_w = total // nw
    nchunks = per_w // chunk
    mesh = plsc.VectorSubcoreMesh(core_axis_name="c", subcore_axis_name="s")

    @functools.partial(
        pl.kernel, mesh=mesh,
        out_type=jax.ShapeDtypeStruct((total, width), jnp.float32),
        scratch_types=[
            pltpu.VMEM((chunk,), jnp.int32),
            pltpu.VMEM((chunk, width), jnp.float32),
            pltpu.SemaphoreType.DMA,
        ],
    )
    def k(table_ref, idx_ref, out_ref, idxv, rows, sem):
        wid = lax.axis_index("s") * info.num_cores + lax.axis_index("c")
        base = wid * per_w

        def body(g, carry):
            off = base + g * chunk
            pltpu.sync_copy(idx_ref.at[pl.ds(off, chunk)], idxv)
            pltpu.async_copy(table_ref.at[idxv], rows, sem).wait()
            pltpu.sync_copy(rows, out_ref.at[pl.ds(off, chunk)])
            return carry

        jax.lax.fori_loop(0, nchunks, body, 0)

    return k(table, idx)


def _k3_body(rows, sqv_ref, sqvp_ref, pq_ref, pk_ref, pkp_ref,
             mk_ref, mkp_ref, o_ref, lse_ref):
    def nrm(t):
        return t / (jnp.sqrt(jnp.sum(t * t, -1, keepdims=True)) + 1e-6)

    for hh in range(rows):
        cur = sqv_ref[hh, 0]
        prv = sqvp_ref[hh, 0]
        cq = cur[:, :DH]
        ck2 = jnp.concatenate([nrm(cq), nrm(prv[:, :DH])], axis=0)
        cv2 = jnp.concatenate([cur[:, DH:], prv[:, DH:]], axis=0)
        dots = jax.lax.dot_general(cq, ck2, (((1,), (1,)), ((), ())),
                                   preferred_element_type=jnp.float32)
        dots = dots * (1.0 / np.sqrt(DH))
        pq = pq_ref[hh, 0]
        pk = jnp.concatenate([pk_ref[hh, 0], pkp_ref[hh, 0]], axis=1)
        dots = dots - 1e5 * (pq == pk).astype(jnp.float32)
        mk = jnp.concatenate([mk_ref[hh, 0], mkp_ref[hh, 0]], axis=1)
        dots = jnp.where(mk != 0, dots, NEG)
        m = jnp.max(dots, -1, keepdims=True)
        ex = jnp.exp(dots - m)
        s = jnp.sum(ex, -1, keepdims=True)
        lse_ref[hh, 0] = m + jnp.log(s)
        o_ref[hh, 0] = jax.lax.dot_general(
            ex / s, cv2, (((1,), (0,)), ((), ())),
            preferred_element_type=jnp.float32)


def _run_k3(sqv, pq, pk, mk, nch):
    rows = sqv.shape[0]
    grid = (nch,)
    prev = lambda i: (0, (i + nch - 1) % nch, 0, 0)
    cur = lambda i: (0, i, 0, 0)
    return pl.pallas_call(
        functools.partial(_k3_body, rows),
        grid=grid,
        in_specs=[
            pl.BlockSpec((rows, 1, BUCKET, 2 * DH), cur),
            pl.BlockSpec((rows, 1, BUCKET, 2 * DH), prev),
            pl.BlockSpec((rows, 1, BUCKET, 1), cur),
            pl.BlockSpec((rows, 1, 1, BUCKET), cur),
            pl.BlockSpec((rows, 1, 1, BUCKET), prev),
            pl.BlockSpec((rows, 1, 1, BUCKET), cur),
            pl.BlockSpec((rows, 1, 1, BUCKET), prev),
        ],
        out_specs=[
            pl.BlockSpec((rows, 1, BUCKET, DH), cur),
            pl.BlockSpec((rows, 1, BUCKET, 1), cur),
        ],
        out_shape=[
            jax.ShapeDtypeStruct((rows, nch, BUCKET, DH), jnp.float32),
            jax.ShapeDtypeStruct((rows, nch, BUCKET, 1), jnp.float32),
        ],
        compiler_params=pltpu.CompilerParams(
            dimension_semantics=("parallel",)),
    )(sqv, sqv, pq, pk, pk, mk, mk)


def _k4_body(bn,
             o0_ref, o1_ref, l0_ref, l1_ref, hpb_ref, e4_ref,
             wo_ref, bo_ref, gps_ref, bps_ref,
             wa_ref, ba_ref, wb_ref, bb_ref, wc_ref, bc_ref,
             wqp_ref, bqp_ref, wkp_ref, bkp_ref, wvp_ref, bvp_ref,
             wop_ref, bop_ref, gpc_ref, bpc_ref,
             wa2_ref, ba2_ref, wb2_ref, bb2_ref, wc2_ref, bc2_ref,
             hob_ref, qoc_ref, wko_ref, bko_ref, wvo_ref, bvo_ref,
             out_ps_ref, out_pc_ref, out_oc_ref,
             m_ps, l_ps, a_ps, m_pc, l_pc, a_pc, m_oc, l_oc, a_oc):
    i = pl.program_id(0)
    nb = pl.num_programs(0)

    @pl.when(i == 0)
    def _init():
        m_ps[...] = jnp.full_like(m_ps[...], -1e30)
        m_pc[...] = jnp.full_like(m_pc[...], -1e30)
        m_oc[...] = jnp.full_like(m_oc[...], -1e30)
        l_ps[...] = jnp.zeros_like(l_ps[...])
        l_pc[...] = jnp.zeros_like(l_pc[...])
        l_oc[...] = jnp.zeros_like(l_oc[...])
        a_ps[...] = jnp.zeros_like(a_ps[...])
        a_pc[...] = jnp.zeros_like(a_pc[...])
        a_oc[...] = jnp.zeros_like(a_oc[...])

    hpb = hpb_ref[...]

    def mm(a, b):
        return jnp.dot(a, b, preferred_element_type=jnp.float32)

    def pool_update(m_r, l_r, a_r, avec, y):
        mb = jnp.max(avec, axis=0, keepdims=True)
        mn = jnp.maximum(m_r[...], mb)
        alpha = jnp.exp(m_r[...] - mn)
        pv = jnp.exp(avec - mn)
        l_r[...] = alpha * l_r[...] + jnp.sum(pv, axis=0, keepdims=True)
        contrib = jax.lax.dot_general(pv, y, (((0,), (0,)), ((), ())),
                                      preferred_element_type=jnp.float32)
        a_r[...] = alpha * a_r[...] + contrib
        m_r[...] = mn

    def gate_branch(x_res, g_r, b_r, wa, ba, wb, bb, wc, bc):
        y = _layernorm(x_res, g_r[...], b_r[...])
        a = jnp.tanh(mm(y, wa[...]) + ba[...])
        s = jax.nn.sigmoid(mm(y, wb[...]) + bb[...])
        avec = mm(a * s, wc[...]) + bc[...]
        return y, avec

    la = l0_ref[...]
    lb = l1_ref[...]
    mml = jnp.maximum(la, lb)
    z = mml + jnp.log(jnp.exp(la - mml) + jnp.exp(lb - mml))
    w0 = mm(jnp.exp(la - z), e4_ref[...])
    w1 = mm(jnp.exp(lb - z), e4_ref[...])
    merged = w0 * o0_ref[...] + w1 * o1_ref[...]
    att = mm(merged, wo_ref[...]) + bo_ref[...]
    y_ps, a_vec = gate_branch(att + hpb, gps_ref, bps_ref,
                              wa_ref, ba_ref, wb_ref, bb_ref, wc_ref, bc_ref)
    pool_update(m_ps, l_ps, a_ps, a_vec, y_ps)

    kp = mm(hob_ref[...], wkp_ref[...]) + bkp_ref[...]
    vp = mm(hob_ref[...], wvp_ref[...]) + bvp_ref[...]
    q = mm(hpb, wqp_ref[...]) + bqp_ref[...]
    colmask = jax.lax.broadcasted_iota(jnp.int32, (1, 8), 1) < 6
    ohs = []
    for hh in range(HEADS):
        sl = slice(hh * DH, (hh + 1) * DH)
        dots = jax.lax.dot_general(q[:, sl], kp[:, sl],
                                   (((1,), (1,)), ((), ())),
                                   preferred_element_type=jnp.float32)
        dots = dots * (1.0 / np.sqrt(DH))
        dots = jnp.where(colmask, dots, NEG)
        mx = jnp.max(dots, -1, keepdims=True)
        ex = jnp.exp(dots - mx)
        attn = ex / jnp.sum(ex, -1, keepdims=True)
        ohs.append(mm(attn, vp[:, sl]))
    o_pc = mm(jnp.concatenate(ohs, axis=-1), wop_ref[...]) + bop_ref[...]
    y_pc, a_vec2 = gate_branch(o_pc + hpb, gpc_ref, bpc_ref, wa2_ref,
                               ba2_ref, wb2_ref, bb2_ref, wc2_ref, bc2_ref)
    pool_update(m_pc, l_pc, a_pc, a_vec2, y_pc)

    ko = mm(hpb, wko_ref[...]) + bko_ref[...]
    vo = mm(hpb, wvo_ref[...]) + bvo_ref[...]
    qoc = qoc_ref[...]
    for hh in range(HEADS):
        sl = slice(hh * DH, (hh + 1) * DH)
        rs = slice(hh * 8, (hh + 1) * 8)
        st = jax.lax.dot_general(qoc[:, sl], ko[:, sl],
                                 (((1,), (1,)), ((), ())),
                                 preferred_element_type=jnp.float32)
        st = st * (1.0 / np.sqrt(DH))
        mb = jnp.max(st, axis=1, keepdims=True)
        mo = m_oc[rs, :]
        mn = jnp.maximum(mo, mb)
        alpha = jnp.exp(mo - mn)
        pmat = jnp.exp(st - mn)
        l_oc[rs, :] = alpha * l_oc[rs, :] + jnp.sum(pmat, axis=1,
                                                    keepdims=True)
        a_oc[rs, :] = alpha * a_oc[rs, :] + mm(pmat, vo[:, sl])
        m_oc[rs, :] = mn

    @pl.when(i == nb - 1)
    def _fin():
        out_ps_ref[...] = a_ps[...] / l_ps[...]
        out_pc_ref[...] = a_pc[...] / l_pc[...]
        out_oc_ref[...] = a_oc[...] / l_oc[...]


def _run_k4(o0, o1, l0, l1, hpb, e4, p, hob8, qoc8, bn):
    n = hpb.shape[0]
    grid = (n // bn,)
    row = lambda i: (i, 0)
    whole = lambda i: (0, 0)

    def wspec(arr):
        return pl.BlockSpec(arr.shape, whole)

    g = p['ps_gate']
    g2 = p['pc_gate']
    ops = [
        o0, o1, l0, l1, hpb, e4,
        p['lsh']['o']['W'], p['lsh']['o']['b'].reshape(1, D),
        p['ln_ps']['g'].reshape(1, D), p['ln_ps']['b'].reshape(1, D),
        g['a']['W'], g['a']['b'].reshape(1, D),
        g['bgate']['W'], g['bgate']['b'].reshape(1, D),
        g['c']['W'], g['c']['b'].reshape(1, 1),
        p['pc_mha']['q']['W'], p['pc_mha']['q']['b'].reshape(1, D),
        p['pc_mha']['k']['W'], p['pc_mha']['k']['b'].reshape(1, D),
        p['pc_mha']['v']['W'], p['pc_mha']['v']['b'].reshape(1, D),
        p['pc_mha']['o']['W'], p['pc_mha']['o']['b'].reshape(1, D),
        p['ln_pc']['g'].reshape(1, D), p['ln_pc']['b'].reshape(1, D),
        g2['a']['W'], g2['a']['b'].reshape(1, D),
        g2['bgate']['W'], g2['bgate']['b'].reshape(1, D),
        g2['c']['W'], g2['c']['b'].reshape(1, 1),
        hob8, qoc8,
        p['oc_mha']['k']['W'], p['oc_mha']['k']['b'].reshape(1, D),
        p['oc_mha']['v']['W'], p['oc_mha']['v']['b'].reshape(1, D),
    ]
    in_specs = [
        pl.BlockSpec((bn, D), row), pl.BlockSpec((bn, D), row),
        pl.BlockSpec((bn, HEADS), row), pl.BlockSpec((bn, HEADS), row),
        pl.BlockSpec((bn, D), row),
    ] + [wspec(a) for a in ops[5:]]
    kfn = functools.partial(_k4_body, bn)
    return pl.pallas_call(
        kfn,
        grid=grid,
        in_specs=in_specs,
        out_specs=[
            pl.BlockSpec((1, D), whole),
            pl.BlockSpec((1, D), whole),
            pl.BlockSpec((HEADS * 8, DH), whole),
        ],
        out_shape=[
            jax.ShapeDtypeStruct((1, D), jnp.float32),
            jax.ShapeDtypeStruct((1, D), jnp.float32),
            jax.ShapeDtypeStruct((HEADS * 8, DH), jnp.float32),
        ],
        scratch_shapes=[
            pltpu.VMEM((1, 1), jnp.float32), pltpu.VMEM((1, 1), jnp.float32),
            pltpu.VMEM((1, D), jnp.float32),
            pltpu.VMEM((1, 1), jnp.float32), pltpu.VMEM((1, 1), jnp.float32),
            pltpu.VMEM((1, D), jnp.float32),
            pltpu.VMEM((HEADS * 8, 1), jnp.float32),
            pltpu.VMEM((HEADS * 8, 1), jnp.float32),
            pltpu.VMEM((HEADS * 8, DH), jnp.float32),
        ],
    )(*ops)


def kernel(x_path, x_omic1, x_omic2, x_omic3, x_omic4, x_omic5, x_omic6,
           params):
    p = params
    n = x_path.shape[0]
    padlen = 2 * BUCKET - n % (2 * BUCKET)
    t = n + padlen
    nch = t // BUCKET
    nbuck = t // BUCKET

    rot = jax.random.normal(jax.random.key(42), (DH, N_HASHES, nbuck // 2))
    rhr = jnp.concatenate([rot, -rot], axis=-1).reshape(DH, N_HASHES * nbuck)
    rmat = jnp.kron(jnp.eye(HEADS, dtype=jnp.float32), rhr)

    bn = 512 if n % 512 == 0 else BUCKET
    hpb, qkvh, bkt = _run_k1(x_path, p, rmat, bn)

    omics = [x_omic1, x_omic2, x_omic3, x_omic4, x_omic5, x_omic6]
    h_omic = [jax.nn.elu(_lin(jax.nn.elu(_lin(o, s['l0'])), s['l1']))
              for o, s in zip(omics, p['sig'])]
    hob = jnp.stack(h_omic)
    hob8 = jnp.concatenate([hob, jnp.zeros((2, D))], axis=0)
    qoc8 = jnp.concatenate(
        [_lin(hob, p['oc_mha']['q']), jnp.zeros((2, D))], axis=0)

    zpad = jnp.zeros((HEADS, padlen, 2 * DH), jnp.float32)
    qkv_t = jnp.concatenate([qkvh, zpad], axis=1)
    bk = bkt.reshape(n, HEADS, N_HASHES).transpose(2, 1, 0)
    bk = jnp.concatenate(
        [bk, jnp.zeros((N_HASHES, HEADS, padlen), jnp.int32)], axis=-1)
    pos = jnp.arange(t, dtype=jnp.int32)
    keys = (bk * t + pos[None, None, :]).reshape(N_HASHES * HEADS, t)
    sidx = jnp.argsort(keys, axis=-1).astype(jnp.int32)
    sidx2 = sidx.reshape(N_HASHES, HEADS, t)
    chunk = 0
    total = N_HASHES * HEADS * t
    if total % 32 == 0:
        per_w = total // 32
        for c in range(128, 0, -8):
            if per_w % c == 0:
                chunk = c
                break
    if chunk:
        table = qkv_t.reshape(HEADS * t, 2 * DH)
        idx_flat = (sidx2
                    + (jnp.arange(HEADS, dtype=jnp.int32)
                       * t)[None, :, None]).reshape(-1)
        sqv = _sc_gather_rows(table, idx_flat, chunk)
    else:
        sqv = jnp.stack(
            [jnp.take_along_axis(qkv_t, sidx2[r][..., None], axis=1)
             for r in range(N_HASHES)], axis=0)
    sqv = sqv.reshape(N_HASHES * HEADS, nch, BUCKET, 2 * DH)
    pq = sidx.reshape(N_HASHES * HEADS, nch, BUCKET, 1)
    pk = sidx.reshape(N_HASHES * HEADS, nch, 1, BUCKET)
    mk = (sidx < n).astype(jnp.int32).reshape(N_HASHES * HEADS, nch, 1,
                                              BUCKET)

    o_s, lse_s = _run_k3(sqv, pq, pk, mk, nch)

    uidx = jnp.argsort(sidx, axis=-1).astype(jnp.int32)
    o_us = jnp.take_along_axis(o_s.reshape(N_HASHES * HEADS, t, DH),
                               uidx[..., None], axis=1)
    l_us = jnp.take_along_axis(lse_s.reshape(N_HASHES * HEADS, t), uidx,
                               axis=1)
    o_us = o_us.reshape(N_HASHES, HEADS, t, DH)[:, :, :n, :]
    l_us = l_us.reshape(N_HASHES, HEADS, t)[:, :, :n]
    o0 = o_us[0].transpose(1, 0, 2).reshape(n, D)
    o1 = o_us[1].transpose(1, 0, 2).reshape(n, D)
    l0 = l_us[0].transpose(1, 0)
    l1 = l_us[1].transpose(1, 0)

    e4 = jnp.kron(jnp.eye(HEADS, dtype=jnp.float32),
                  jnp.ones((1, DH), jnp.float32))

    pooled_ps, pooled_pc, oc_acc = _run_k4(o0, o1, l0, l1, hpb, e4, p,
                                           hob8, qoc8, bn)

    hps = jax.nn.relu(_lin(pooled_ps, p['ps_rho']))
    hpc = jax.nn.relu(_lin(pooled_pc, p['pc_rho']))

    oc_h = oc_acc.reshape(HEADS, 8, DH)[:, :6, :]
    oc_m = oc_h.transpose(1, 0, 2).reshape(6, D)
    hoc = _lin(oc_m, p['oc_mha']['o'])
    hoc = _layernorm(hoc + hob, p['ln_oc']['g'], p['ln_oc']['b'])
    a = jnp.tanh(_lin(hoc, p['oc_gate']['a']))
    b = jax.nn.sigmoid(_lin(hoc, p['oc_gate']['bgate']))
    av = _lin(a * b, p['oc_gate']['c'])
    hoc = jax.nn.softmax(av.T, 1) @ hoc
    hoc = jax.nn.relu(_lin(hoc, p['oc_rho']))

    def mha6(q, k, v, mp):
        qp = _lin(q, mp['q']); kp = _lin(k, mp['k']); vp = _lin(v, mp['v'])
        qh = qp.reshape(6, HEADS, DH).transpose(1, 0, 2)
        kh = kp.reshape(6, HEADS, DH).transpose(1, 0, 2)
        vh_ = vp.reshape(6, HEADS, DH).transpose(1, 0, 2)
        att = jax.nn.softmax(
            jnp.einsum('hid,hjd->hij', qh, kh) / np.sqrt(DH), -1)
        o = jnp.einsum('hij,hjd->hid', att, vh_).transpose(1, 0, 2)
        return _lin(o.reshape(6, D), mp['o'])

    hos = mha6(hob, hob, hob, p['os_mha'])
    hos = _layernorm(hos + hob, p['ln_os']['g'], p['ln_os']['b'])
    a = jnp.tanh(_lin(hos, p['os_gate']['a']))
    b = jax.nn.sigmoid(_lin(hos, p['os_gate']['bgate']))
    av = _lin(a * b, p['os_gate']['c'])
    hos = jax.nn.softmax(av.T, 1) @ hos
    hos = jax.nn.relu(_lin(hos, p['os_rho']))

    h = jnp.concatenate([hps, hpc, hos, hoc], 0)
    qs = _lin(h, p['query'])
    ks = _lin(h, p['key'])
    aw = jax.nn.softmax(qs @ ks.T, 1)
    hf = (aw @ h).sum(0)
    logits = _lin(hf, p['cls'])[None]
    return (logits, _lin(hps, p['cls_ps']), _lin(hpc, p['cls_pc']),
            _lin(hos, p['cls_os']), _lin(hoc, p['cls_oc']))
